# Initial kernel scaffold; baseline (speedup 1.0000x reference)
import jax
import jax.numpy as jnp
from jax.experimental import pallas as pl


def kernel(params, x, edge_index, edge_attr, initial_state):
    raise NotImplementedError("write your pallas kernel here")


# speedup vs baseline: 2.2956x; 2.2956x over previous
import functools

import jax
import jax.numpy as jnp
from jax import lax
from jax.experimental import pallas as pl
from jax.experimental.pallas import tpu as pltpu
from jax.experimental.pallas import tpu_sc as plsc

F32 = jnp.float32
NN = 10000
NE = 160000
CH = 128
NCHUNK = NE // CH
NWORK = 32
NNP = 10240
SLAB = NNP // 16

_MESH = plsc.VectorSubcoreMesh(core_axis_name="c", subcore_axis_name="s")


def _sc_segsum(table, src, dst, rows=None):
    T, W = table.shape
    Wh = W // 2
    table2 = table.reshape(T * 2, Wh)
    has_rows = rows is not None

    def body(*refs):
        if has_rows:
            (tab_r, src_r, dst_r, rows_r, zer_r, out_r,
             src_v, dst_v, idx_v, grow_v, erow_v, sem, shared) = refs
        else:
            (tab_r, src_r, dst_r, zer_r, out_r,
             src_v, dst_v, idx_v, grow_v, sem, shared) = refs
            rows_r = erow_v = None
        cid = lax.axis_index("c")
        sid = lax.axis_index("s")
        base = sid * SLAB

        pltpu.sync_copy(zer_r, grow_v)
        for r in range(SLAB // CH):
            pltpu.sync_copy(grow_v, shared.at[pl.ds(base + r * CH, CH)])
        plsc.subcore_barrier()

        nsteps = (NCHUNK - sid + 15) // 16

        def step(i, carry):
            e0 = pl.multiple_of((sid + 16 * i) * CH, 8)
            pltpu.sync_copy(src_r.at[pl.ds(e0, CH)], src_v)
            pltpu.sync_copy(dst_r.at[pl.ds(e0, CH)], dst_v)
            for j in range(CH // 16):
                s = src_v[pl.ds(j * 16, 16)]
                idx_v[pl.ds(j * 16, 16)] = s + s + cid
            pltpu.async_copy(tab_r.at[idx_v], grow_v, sem).wait()
            pltpu.sync_copy(grow_v, shared.at[dst_v], add=True)
            if has_rows:
                pltpu.sync_copy(
                    rows_r.at[pl.ds(e0, CH), pl.ds(cid * Wh, Wh)], erow_v)
                pltpu.sync_copy(erow_v, shared.at[dst_v], add=True)
            return carry

        lax.fori_loop(0, nsteps, step, 0)
        plsc.subcore_barrier()
        pltpu.sync_copy(shared.at[pl.ds(base, SLAB)],
                        out_r.at[pl.ds(base, SLAB), pl.ds(cid * Wh, Wh)])

    scratch = [
        pltpu.VMEM((CH,), jnp.int32),
        pltpu.VMEM((CH,), jnp.int32),
        pltpu.VMEM((CH,), jnp.int32),
        pltpu.VMEM((CH, Wh), F32),
    ]
    if has_rows:
        scratch.append(pltpu.VMEM((CH, Wh), F32))
    scratch += [pltpu.SemaphoreType.DMA, pltpu.VMEM_SHARED((NNP, Wh), F32)]

    zer = jnp.zeros((CH, Wh), F32)
    kern = pl.kernel(
        body,
        out_type=jax.ShapeDtypeStruct((NNP, W), F32),
        mesh=_MESH,
        scratch_types=scratch,
    )
    if has_rows:
        return kern(table2, src, dst, rows, zer)
    return kern(table2, src, dst, zer)


def _bn(n):
    return 400 if n == NN else 1000


def _tc_pre(x, w, scale):
    n, k = x.shape
    wo = w.shape[1]
    bn = _bn(n)

    def body(x_r, w_r, s_r, o_r):
        o_r[...] = jnp.dot(x_r[...], w_r[...],
                           preferred_element_type=F32) * s_r[...]

    return pl.pallas_call(
        body,
        grid=(n // bn,),
        in_specs=[
            pl.BlockSpec((bn, k), lambda i: (i, 0)),
            pl.BlockSpec((k, wo), lambda i: (0, 0)),
            pl.BlockSpec((bn, 1), lambda i: (i, 0)),
        ],
        out_specs=pl.BlockSpec((bn, wo), lambda i: (i, 0)),
        out_shape=jax.ShapeDtypeStruct((n, wo), F32),
    )(x, w, scale.reshape(n, 1))


def _tc_mid(s, g, scale, b, w2, mask_n=None):
    n, w = g.shape
    wo = w2.shape[1]
    bn = _bn(n)
    nbs = (mask_n // bn) if mask_n is not None else None

    def body(s_r, g_r, sc_r, b_r, w2_r, o_r):
        sm = s_r[...]
        if mask_n is not None:
            i = pl.program_id(0)
            row = i * bn + lax.broadcasted_iota(jnp.int32, (bn, 1), 0)
            sm = jnp.where(row < mask_n, sm, 0.0)
        h = jnp.maximum(sc_r[...] * (sm + g_r[...]) + b_r[...], 0.0)
        o_r[...] = jnp.dot(h, w2_r[...],
                           preferred_element_type=F32) * sc_r[...]

    s_map = ((lambda i: (jnp.minimum(i, nbs - 1), 0)) if mask_n is not None
             else (lambda i: (i, 0)))
    return pl.pallas_call(
        body,
        grid=(n // bn,),
        in_specs=[
            pl.BlockSpec((bn, w), s_map),
            pl.BlockSpec((bn, w), lambda i: (i, 0)),
            pl.BlockSpec((bn, 1), lambda i: (i, 0)),
            pl.BlockSpec((1, w), lambda i: (0, 0)),
            pl.BlockSpec((w, wo), lambda i: (0, 0)),
        ],
        out_specs=pl.BlockSpec((bn, wo), lambda i: (i, 0)),
        out_shape=jax.ShapeDtypeStruct((n, wo), F32),
    )(s, g, scale.reshape(n, 1), b.reshape(1, w), w2)


def _tc_post(s, g, scale, b, mask_n=None):
    n, w = g.shape
    bn = _bn(n)
    nbs = (mask_n // bn) if mask_n is not None else None

    def body(s_r, g_r, sc_r, b_r, o_r):
        sm = s_r[...]
        if mask_n is not None:
            i = pl.program_id(0)
            row = i * bn + lax.broadcasted_iota(jnp.int32, (bn, 1), 0)
            sm = jnp.where(row < mask_n, sm, 0.0)
        o_r[...] = sc_r[...] * (sm + g_r[...]) + b_r[...]

    s_map = ((lambda i: (jnp.minimum(i, nbs - 1), 0)) if mask_n is not None
             else (lambda i: (i, 0)))
    return pl.pallas_call(
        body,
        grid=(n // bn,),
        in_specs=[
            pl.BlockSpec((bn, w), s_map),
            pl.BlockSpec((bn, w), lambda i: (i, 0)),
            pl.BlockSpec((bn, 1), lambda i: (i, 0)),
            pl.BlockSpec((1, w), lambda i: (0, 0)),
        ],
        out_specs=pl.BlockSpec((bn, w), lambda i: (i, 0)),
        out_shape=jax.ShapeDtypeStruct((n, w), F32),
    )(s, g, scale.reshape(n, 1), b.reshape(1, w))


def _tc_mlp(x, p, res=None, n=None):
    k = x.shape[1]
    n = x.shape[0] if n is None else n
    dh = p["W1"].shape[1]
    wo = p["W2"].shape[1]
    bn = _bn(n)

    def body(x_r, w1_r, b1_r, w2_r, b2_r, g_r, be_r, *rest):
        if res is not None:
            r_r, o_r = rest
        else:
            (o_r,) = rest
        h = jnp.maximum(jnp.dot(x_r[...], w1_r[...],
                                preferred_element_type=F32) + b1_r[...], 0.0)
        y = jnp.dot(h, w2_r[...], preferred_element_type=F32) + b2_r[...]
        mu = jnp.mean(y, axis=-1, keepdims=True)
        yc = y - mu
        var = jnp.mean(yc * yc, axis=-1, keepdims=True)
        o = yc * lax.rsqrt(var + 1e-5) * g_r[...] + be_r[...]
        if res is not None:
            o = o + r_r[...]
        o_r[...] = o

    in_specs = [
        pl.BlockSpec((bn, k), lambda i: (i, 0)),
        pl.BlockSpec((k, dh), lambda i: (0, 0)),
        pl.BlockSpec((1, dh), lambda i: (0, 0)),
        pl.BlockSpec((dh, wo), lambda i: (0, 0)),
        pl.BlockSpec((1, wo), lambda i: (0, 0)),
        pl.BlockSpec((1, wo), lambda i: (0, 0)),
        pl.BlockSpec((1, wo), lambda i: (0, 0)),
    ]
    args = [x, p["W1"], p["b1"].reshape(1, dh), p["W2"],
            p["b2"].reshape(1, wo), p["g"].reshape(1, wo),
            p["beta"].reshape(1, wo)]
    if res is not None:
        in_specs.append(pl.BlockSpec((bn, wo), lambda i: (i, 0)))
        args.append(res)
    return pl.pallas_call(
        body,
        grid=(n // bn,),
        in_specs=in_specs,
        out_specs=pl.BlockSpec((bn, wo), lambda i: (i, 0)),
        out_shape=jax.ShapeDtypeStruct((n, wo), F32),
    )(*args)


def _tc_dec(s6, g2, scale, b2p, init, init3, fcwp, fcw8, fcb):
    n = g2.shape[0]
    bn = _bn(n)

    def body(s_r, g_r, sc_r, b_r, i_r, i3_r, wp_r, w8_r, fb_r, o_r):
        d = sc_r[...] * (s_r[...] + g_r[...]) + b_r[...]
        o_r[...] = (i3_r[...] + fb_r[...]
                    + jnp.dot(d, wp_r[...], preferred_element_type=F32)
                    + jnp.dot(i_r[...], w8_r[...],
                              preferred_element_type=F32))

    return pl.pallas_call(
        body,
        grid=(n // bn,),
        in_specs=[
            pl.BlockSpec((bn, 256), lambda i: (i, 0)),
            pl.BlockSpec((bn, 256), lambda i: (i, 0)),
            pl.BlockSpec((bn, 1), lambda i: (i, 0)),
            pl.BlockSpec((1, 256), lambda i: (0, 0)),
            pl.BlockSpec((bn, 8), lambda i: (i, 0)),
            pl.BlockSpec((bn, 3), lambda i: (i, 0)),
            pl.BlockSpec((256, 3), lambda i: (0, 0)),
            pl.BlockSpec((8, 3), lambda i: (0, 0)),
            pl.BlockSpec((1, 3), lambda i: (0, 0)),
        ],
        out_specs=pl.BlockSpec((bn, 3), lambda i: (i, 0)),
        out_shape=jax.ShapeDtypeStruct((n, 3), F32),
    )(s6, g2, scale.reshape(n, 1), b2p.reshape(1, 256), init, init3,
      fcwp, fcw8, fcb.reshape(1, 3))


def kernel(params, x, edge_index, edge_attr, initial_state):
    src = edge_index[0]
    dst = edge_index[1]

    cnt = _sc_segsum(jnp.ones((8, 256), F32), jnp.zeros_like(src), dst)
    deg = cnt[:NN, 0] + 1.0
    dinv_n = lax.rsqrt(deg)
    dinv_e = jnp.concatenate([dinv_n, jnp.ones((NE - NN,), F32)])

    en = params["enc_node"]
    g1 = _tc_pre(x, en["W1"], dinv_n)
    s1 = _sc_segsum(g1, src, dst)
    g2 = _tc_mid(s1, g1, dinv_n, en["b1"], en["W2"])
    s2 = _sc_segsum(g2, src, dst)
    h = _tc_post(s2, g2, dinv_n, en["b2"])

    ee = params["enc_edge"]
    a1 = _tc_pre(edge_attr, ee["W1"], dinv_e)
    s3 = _sc_segsum(a1, src, dst)
    a2 = _tc_mid(s3, a1, dinv_e, ee["b1"], ee["W2"], mask_n=NN)
    s4 = _sc_segsum(a2, src, dst)
    ea = _tc_post(s4, a2, dinv_e, ee["b2"], mask_n=NN)

    for lp in params["proc"]:
        e_rows = _tc_mlp(ea, lp["edge_mlp"])
        aggr = _sc_segsum(h, src, dst, rows=e_rows)
        h = _tc_mlp(aggr, lp["node_mlp"], res=h, n=NN)

    dc = params["dec"]
    gd1 = _tc_pre(h, dc["W1"], dinv_n)
    s5 = _sc_segsum(gd1, src, dst)
    w2p = jnp.pad(dc["W2"], ((0, 0), (0, 253)))
    gd2 = _tc_mid(s5, gd1, dinv_n, dc["b1"], w2p)
    s6 = _sc_segsum(gd2, src, dst)
    b2p = jnp.pad(dc["b2"], (0, 253))
    fcwp = jnp.pad(dc["fcW"][:3], ((0, 253), (0, 0)))
    return _tc_dec(s6, gd2, dinv_n, b2p, initial_state,
                   initial_state[:, :3], fcwp, dc["fcW"][3:], dc["fcb"])


# speedup vs baseline: 2.5998x; 1.1325x over previous
import functools

import jax
import jax.numpy as jnp
from jax import lax
from jax.experimental import pallas as pl
from jax.experimental.pallas import tpu as pltpu
from jax.experimental.pallas import tpu_sc as plsc

F32 = jnp.float32
NN = 10000
NE = 160000
CH = 128
NCHUNK = NE // CH
NWORK = 32
NNP = 10112
SLAB = NNP // 16

_MESH = plsc.VectorSubcoreMesh(core_axis_name="c", subcore_axis_name="s")


def _sc_segsum(table, src, dst, rows=None):
    T, W = table.shape
    Wh = W // 2
    table2 = table.reshape(T * 2, Wh)
    has_rows = rows is not None

    U = 2

    def body(*refs):
        if has_rows:
            (tab_r, src_r, dst_r, rows_r, zer_r, out_r, src_v, dst_v, idx_v,
             grow_v, erow_v, lsem, gsem, rsem, shared) = refs
        else:
            (tab_r, src_r, dst_r, zer_r, out_r, src_v, dst_v, idx_v,
             grow_v, lsem, gsem, shared) = refs
            rows_r = erow_v = rsem = None
        cid = lax.axis_index("c")
        sid = lax.axis_index("s")
        base = sid * SLAB

        pltpu.sync_copy(zer_r, grow_v.at[0])
        for r in range((SLAB + CH - 1) // CH):
            hh = min(CH, SLAB - r * CH)
            pltpu.sync_copy(grow_v.at[0, pl.ds(0, hh)],
                            shared.at[pl.ds(base + r * CH, hh)])
        plsc.subcore_barrier()

        nsteps = (NCHUNK - sid + 15) // 16

        def chunk_off(s):
            return pl.multiple_of((sid + 16 * s) * CH, 8)

        def stepu(i, carry):
            lds = []
            for u in range(U):
                e0 = chunk_off(U * i + u)
                lds.append(pltpu.async_copy(
                    src_r.at[pl.ds(e0, CH)], src_v.at[u], lsem))
                lds.append(pltpu.async_copy(
                    dst_r.at[pl.ds(e0, CH)], dst_v.at[u], lsem))
            for d in lds:
                d.wait()
            gds = []
            for u in range(U):
                for j in range(CH // 16):
                    s = src_v[u, pl.ds(j * 16, 16)]
                    idx_v[u, pl.ds(j * 16, 16)] = s + s + cid
                gds.append(pltpu.async_copy(
                    tab_r.at[idx_v.at[u]], grow_v.at[u], gsem))
            for u in range(U):
                if has_rows:
                    e0 = chunk_off(U * i + u)
                    rd = pltpu.async_copy(
                        rows_r.at[pl.ds(e0, CH), pl.ds(cid * Wh, Wh)],
                        erow_v, rsem)
                gds[u].wait()
                pltpu.sync_copy(grow_v.at[u], shared.at[dst_v.at[u]],
                                add=True)
                if has_rows:
                    rd.wait()
                    pltpu.sync_copy(erow_v, shared.at[dst_v.at[u]],
                                    add=True)
            return carry

        def step1(s, carry):
            e0 = chunk_off(s)
            pltpu.sync_copy(src_r.at[pl.ds(e0, CH)], src_v.at[0])
            pltpu.sync_copy(dst_r.at[pl.ds(e0, CH)], dst_v.at[0])
            for j in range(CH // 16):
                ss = src_v[0, pl.ds(j * 16, 16)]
                idx_v[0, pl.ds(j * 16, 16)] = ss + ss + cid
            gd = pltpu.async_copy(tab_r.at[idx_v.at[0]], grow_v.at[0], gsem)
            if has_rows:
                rd = pltpu.async_copy(
                    rows_r.at[pl.ds(e0, CH), pl.ds(cid * Wh, Wh)],
                    erow_v, rsem)
            gd.wait()
            pltpu.sync_copy(grow_v.at[0], shared.at[dst_v.at[0]], add=True)
            if has_rows:
                rd.wait()
                pltpu.sync_copy(erow_v, shared.at[dst_v.at[0]], add=True)
            return carry

        nfull = nsteps // U
        lax.fori_loop(0, nfull, stepu, 0)
        lax.fori_loop(U * nfull, nsteps, step1, 0)
        plsc.subcore_barrier()
        pltpu.sync_copy(shared.at[pl.ds(base, SLAB)],
                        out_r.at[pl.ds(base, SLAB), pl.ds(cid * Wh, Wh)])

    scratch = [
        pltpu.VMEM((U, CH), jnp.int32),
        pltpu.VMEM((U, CH), jnp.int32),
        pltpu.VMEM((U, CH), jnp.int32),
        pltpu.VMEM((U, CH, Wh), F32),
    ]
    if has_rows:
        scratch.append(pltpu.VMEM((CH, Wh), F32))
    scratch += [pltpu.SemaphoreType.DMA, pltpu.SemaphoreType.DMA]
    if has_rows:
        scratch.append(pltpu.SemaphoreType.DMA)
    scratch.append(pltpu.VMEM_SHARED((NNP, Wh), F32))

    zer = jnp.zeros((CH, Wh), F32)
    kern = pl.kernel(
        body,
        out_type=jax.ShapeDtypeStruct((NNP, W), F32),
        mesh=_MESH,
        scratch_types=scratch,
    )
    if has_rows:
        return kern(table2, src, dst, rows, zer)
    return kern(table2, src, dst, zer)


def _bn(n):
    return 400 if n == NN else 1000


def _tc_pre(x, w, scale):
    n, k = x.shape
    wo = w.shape[1]
    bn = _bn(n)

    def body(x_r, w_r, s_r, o_r):
        o_r[...] = jnp.dot(x_r[...], w_r[...],
                           preferred_element_type=F32) * s_r[...]

    return pl.pallas_call(
        body,
        grid=(n // bn,),
        in_specs=[
            pl.BlockSpec((bn, k), lambda i: (i, 0)),
            pl.BlockSpec((k, wo), lambda i: (0, 0)),
            pl.BlockSpec((bn, 1), lambda i: (i, 0)),
        ],
        out_specs=pl.BlockSpec((bn, wo), lambda i: (i, 0)),
        out_shape=jax.ShapeDtypeStruct((n, wo), F32),
    )(x, w, scale.reshape(n, 1))


def _tc_mid(s, g, scale, b, w2, mask_n=None):
    n, w = g.shape
    wo = w2.shape[1]
    bn = _bn(n)
    nbs = (mask_n // bn) if mask_n is not None else None

    def body(s_r, g_r, sc_r, b_r, w2_r, o_r):
        sm = s_r[...]
        if mask_n is not None:
            i = pl.program_id(0)
            row = i * bn + lax.broadcasted_iota(jnp.int32, (bn, 1), 0)
            sm = jnp.where(row < mask_n, sm, 0.0)
        h = jnp.maximum(sc_r[...] * (sm + g_r[...]) + b_r[...], 0.0)
        o_r[...] = jnp.dot(h, w2_r[...],
                           preferred_element_type=F32) * sc_r[...]

    s_map = ((lambda i: (jnp.minimum(i, nbs - 1), 0)) if mask_n is not None
             else (lambda i: (i, 0)))
    return pl.pallas_call(
        body,
        grid=(n // bn,),
        in_specs=[
            pl.BlockSpec((bn, w), s_map),
            pl.BlockSpec((bn, w), lambda i: (i, 0)),
            pl.BlockSpec((bn, 1), lambda i: (i, 0)),
            pl.BlockSpec((1, w), lambda i: (0, 0)),
            pl.BlockSpec((w, wo), lambda i: (0, 0)),
        ],
        out_specs=pl.BlockSpec((bn, wo), lambda i: (i, 0)),
        out_shape=jax.ShapeDtypeStruct((n, wo), F32),
    )(s, g, scale.reshape(n, 1), b.reshape(1, w), w2)


def _tc_post(s, g, scale, b, mask_n=None):
    n, w = g.shape
    bn = _bn(n)
    nbs = (mask_n // bn) if mask_n is not None else None

    def body(s_r, g_r, sc_r, b_r, o_r):
        sm = s_r[...]
        if mask_n is not None:
            i = pl.program_id(0)
            row = i * bn + lax.broadcasted_iota(jnp.int32, (bn, 1), 0)
            sm = jnp.where(row < mask_n, sm, 0.0)
        o_r[...] = sc_r[...] * (sm + g_r[...]) + b_r[...]

    s_map = ((lambda i: (jnp.minimum(i, nbs - 1), 0)) if mask_n is not None
             else (lambda i: (i, 0)))
    return pl.pallas_call(
        body,
        grid=(n // bn,),
        in_specs=[
            pl.BlockSpec((bn, w), s_map),
            pl.BlockSpec((bn, w), lambda i: (i, 0)),
            pl.BlockSpec((bn, 1), lambda i: (i, 0)),
            pl.BlockSpec((1, w), lambda i: (0, 0)),
        ],
        out_specs=pl.BlockSpec((bn, w), lambda i: (i, 0)),
        out_shape=jax.ShapeDtypeStruct((n, w), F32),
    )(s, g, scale.reshape(n, 1), b.reshape(1, w))


def _tc_mlp(x, p, res=None, n=None):
    k = x.shape[1]
    n = x.shape[0] if n is None else n
    dh = p["W1"].shape[1]
    wo = p["W2"].shape[1]
    bn = _bn(n)

    def body(x_r, w1_r, b1_r, w2_r, b2_r, g_r, be_r, *rest):
        if res is not None:
            r_r, o_r = rest
        else:
            (o_r,) = rest
        h = jnp.maximum(jnp.dot(x_r[...], w1_r[...],
                                preferred_element_type=F32) + b1_r[...], 0.0)
        y = jnp.dot(h, w2_r[...], preferred_element_type=F32) + b2_r[...]
        mu = jnp.mean(y, axis=-1, keepdims=True)
        yc = y - mu
        var = jnp.mean(yc * yc, axis=-1, keepdims=True)
        o = yc * lax.rsqrt(var + 1e-5) * g_r[...] + be_r[...]
        if res is not None:
            o = o + r_r[...]
        o_r[...] = o

    in_specs = [
        pl.BlockSpec((bn, k), lambda i: (i, 0)),
        pl.BlockSpec((k, dh), lambda i: (0, 0)),
        pl.BlockSpec((1, dh), lambda i: (0, 0)),
        pl.BlockSpec((dh, wo), lambda i: (0, 0)),
        pl.BlockSpec((1, wo), lambda i: (0, 0)),
        pl.BlockSpec((1, wo), lambda i: (0, 0)),
        pl.BlockSpec((1, wo), lambda i: (0, 0)),
    ]
    args = [x, p["W1"], p["b1"].reshape(1, dh), p["W2"],
            p["b2"].reshape(1, wo), p["g"].reshape(1, wo),
            p["beta"].reshape(1, wo)]
    if res is not None:
        in_specs.append(pl.BlockSpec((bn, wo), lambda i: (i, 0)))
        args.append(res)
    return pl.pallas_call(
        body,
        grid=(n // bn,),
        in_specs=in_specs,
        out_specs=pl.BlockSpec((bn, wo), lambda i: (i, 0)),
        out_shape=jax.ShapeDtypeStruct((n, wo), F32),
    )(*args)


def _tc_dec(s6, g2, scale, b2p, init, init3, fcwp, fcw8, fcb):
    n = g2.shape[0]
    bn = _bn(n)

    def body(s_r, g_r, sc_r, b_r, i_r, i3_r, wp_r, w8_r, fb_r, o_r):
        d = sc_r[...] * (s_r[...] + g_r[...]) + b_r[...]
        o_r[...] = (i3_r[...] + fb_r[...]
                    + jnp.dot(d, wp_r[...], preferred_element_type=F32)
                    + jnp.dot(i_r[...], w8_r[...],
                              preferred_element_type=F32))

    return pl.pallas_call(
        body,
        grid=(n // bn,),
        in_specs=[
            pl.BlockSpec((bn, 256), lambda i: (i, 0)),
            pl.BlockSpec((bn, 256), lambda i: (i, 0)),
            pl.BlockSpec((bn, 1), lambda i: (i, 0)),
            pl.BlockSpec((1, 256), lambda i: (0, 0)),
            pl.BlockSpec((bn, 8), lambda i: (i, 0)),
            pl.BlockSpec((bn, 3), lambda i: (i, 0)),
            pl.BlockSpec((256, 3), lambda i: (0, 0)),
            pl.BlockSpec((8, 3), lambda i: (0, 0)),
            pl.BlockSpec((1, 3), lambda i: (0, 0)),
        ],
        out_specs=pl.BlockSpec((bn, 3), lambda i: (i, 0)),
        out_shape=jax.ShapeDtypeStruct((n, 3), F32),
    )(s6, g2, scale.reshape(n, 1), b2p.reshape(1, 256), init, init3,
      fcwp, fcw8, fcb.reshape(1, 3))


def kernel(params, x, edge_index, edge_attr, initial_state):
    src = edge_index[0]
    dst = edge_index[1]

    cnt = _sc_segsum(jnp.ones((8, 256), F32), jnp.zeros_like(src), dst)
    deg = cnt[:NN, 0] + 1.0
    dinv_n = lax.rsqrt(deg)
    dinv_e = jnp.concatenate([dinv_n, jnp.ones((NE - NN,), F32)])

    en = params["enc_node"]
    g1 = _tc_pre(x, en["W1"], dinv_n)
    s1 = _sc_segsum(g1, src, dst)
    g2 = _tc_mid(s1, g1, dinv_n, en["b1"], en["W2"])
    s2 = _sc_segsum(g2, src, dst)
    h = _tc_post(s2, g2, dinv_n, en["b2"])

    ee = params["enc_edge"]
    a1 = _tc_pre(edge_attr, ee["W1"], dinv_e)
    s3 = _sc_segsum(a1, src, dst)
    a2 = _tc_mid(s3, a1, dinv_e, ee["b1"], ee["W2"], mask_n=NN)
    s4 = _sc_segsum(a2, src, dst)
    ea = _tc_post(s4, a2, dinv_e, ee["b2"], mask_n=NN)

    for lp in params["proc"]:
        e_rows = _tc_mlp(ea, lp["edge_mlp"])
        aggr = _sc_segsum(h, src, dst, rows=e_rows)
        h = _tc_mlp(aggr, lp["node_mlp"], res=h, n=NN)

    dc = params["dec"]
    gd1 = _tc_pre(h, dc["W1"], dinv_n)
    s5 = _sc_segsum(gd1, src, dst)
    w2p = jnp.pad(dc["W2"], ((0, 0), (0, 253)))
    gd2 = _tc_mid(s5, gd1, dinv_n, dc["b1"], w2p)
    s6 = _sc_segsum(gd2, src, dst)
    b2p = jnp.pad(dc["b2"], (0, 253))
    fcwp = jnp.pad(dc["fcW"][:3], ((0, 253), (0, 0)))
    return _tc_dec(s6, gd2, dinv_n, b2p, initial_state,
                   initial_state[:, :3], fcwp, dc["fcW"][3:], dc["fcb"])


# speedup vs baseline: 5.5972x; 2.1529x over previous
import functools

import jax
import jax.numpy as jnp
from jax import lax
from jax.experimental import pallas as pl
from jax.experimental.pallas import tpu as pltpu
from jax.experimental.pallas import tpu_sc as plsc

F32 = jnp.float32
NN = 10000
NE = 160000
CH = 128
NCHUNK = NE // CH
NWORK = 32
NNP = 10112
SLAB = NNP // 16

_MESH = plsc.VectorSubcoreMesh(core_axis_name="c", subcore_axis_name="s")


def _sc_segsum(table, src, dst, rows=None):
    T, W = table.shape
    Wh = W // 2
    table2 = table.reshape(T * 2, Wh)
    has_rows = rows is not None

    U = 2

    def body(*refs):
        if has_rows:
            (tab_r, src_r, dst_r, rows_r, zer_r, out_r, src_v, dst_v, idx_v,
             grow_v, erow_v, lsem, gsem, rsem, shared) = refs
        else:
            (tab_r, src_r, dst_r, zer_r, out_r, src_v, dst_v, idx_v,
             grow_v, lsem, gsem, shared) = refs
            rows_r = erow_v = rsem = None
        cid = lax.axis_index("c")
        sid = lax.axis_index("s")
        base = sid * SLAB

        pltpu.sync_copy(zer_r, grow_v.at[0])
        for r in range((SLAB + CH - 1) // CH):
            hh = min(CH, SLAB - r * CH)
            pltpu.sync_copy(grow_v.at[0, pl.ds(0, hh)],
                            shared.at[pl.ds(base + r * CH, hh)])
        plsc.subcore_barrier()

        nsteps = (NCHUNK - sid + 15) // 16

        def chunk_off(s):
            return pl.multiple_of((sid + 16 * s) * CH, 8)

        def stepu(i, carry):
            lds = []
            for u in range(U):
                e0 = chunk_off(U * i + u)
                lds.append(pltpu.async_copy(
                    src_r.at[pl.ds(e0, CH)], src_v.at[u], lsem))
                lds.append(pltpu.async_copy(
                    dst_r.at[pl.ds(e0, CH)], dst_v.at[u], lsem))
            for d in lds:
                d.wait()
            gds = []
            for u in range(U):
                for j in range(CH // 16):
                    s = src_v[u, pl.ds(j * 16, 16)]
                    idx_v[u, pl.ds(j * 16, 16)] = s + s + cid
                gds.append(pltpu.async_copy(
                    tab_r.at[idx_v.at[u]], grow_v.at[u], gsem))
            for u in range(U):
                if has_rows:
                    e0 = chunk_off(U * i + u)
                    rd = pltpu.async_copy(
                        rows_r.at[pl.ds(e0, CH), pl.ds(cid * Wh, Wh)],
                        erow_v, rsem)
                gds[u].wait()
                pltpu.sync_copy(grow_v.at[u], shared.at[dst_v.at[u]],
                                add=True)
                if has_rows:
                    rd.wait()
                    pltpu.sync_copy(erow_v, shared.at[dst_v.at[u]],
                                    add=True)
            return carry

        def step1(s, carry):
            e0 = chunk_off(s)
            pltpu.sync_copy(src_r.at[pl.ds(e0, CH)], src_v.at[0])
            pltpu.sync_copy(dst_r.at[pl.ds(e0, CH)], dst_v.at[0])
            for j in range(CH // 16):
                ss = src_v[0, pl.ds(j * 16, 16)]
                idx_v[0, pl.ds(j * 16, 16)] = ss + ss + cid
            gd = pltpu.async_copy(tab_r.at[idx_v.at[0]], grow_v.at[0], gsem)
            if has_rows:
                rd = pltpu.async_copy(
                    rows_r.at[pl.ds(e0, CH), pl.ds(cid * Wh, Wh)],
                    erow_v, rsem)
            gd.wait()
            pltpu.sync_copy(grow_v.at[0], shared.at[dst_v.at[0]], add=True)
            if has_rows:
                rd.wait()
                pltpu.sync_copy(erow_v, shared.at[dst_v.at[0]], add=True)
            return carry

        nfull = nsteps // U
        lax.fori_loop(0, nfull, stepu, 0)
        lax.fori_loop(U * nfull, nsteps, step1, 0)
        plsc.subcore_barrier()
        pltpu.sync_copy(shared.at[pl.ds(base, SLAB)],
                        out_r.at[pl.ds(base, SLAB), pl.ds(cid * Wh, Wh)])

    scratch = [
        pltpu.VMEM((U, CH), jnp.int32),
        pltpu.VMEM((U, CH), jnp.int32),
        pltpu.VMEM((U, CH), jnp.int32),
        pltpu.VMEM((U, CH, Wh), F32),
    ]
    if has_rows:
        scratch.append(pltpu.VMEM((CH, Wh), F32))
    scratch += [pltpu.SemaphoreType.DMA, pltpu.SemaphoreType.DMA]
    if has_rows:
        scratch.append(pltpu.SemaphoreType.DMA)
    scratch.append(pltpu.VMEM_SHARED((NNP, Wh), F32))

    zer = jnp.zeros((CH, Wh), F32)
    kern = pl.kernel(
        body,
        out_type=jax.ShapeDtypeStruct((NNP, W), F32),
        mesh=_MESH,
        scratch_types=scratch,
    )
    if has_rows:
        return kern(table2, src, dst, rows, zer)
    return kern(table2, src, dst, zer)


def _bn(n):
    return 400 if n == NN else 1000


def _tc_pre(x, w, scale):
    n, k = x.shape
    wo = w.shape[1]
    bn = _bn(n)

    def body(x_r, w_r, s_r, o_r):
        o_r[...] = jnp.dot(x_r[...], w_r[...],
                           preferred_element_type=F32) * s_r[...]

    return pl.pallas_call(
        body,
        grid=(n // bn,),
        in_specs=[
            pl.BlockSpec((bn, k), lambda i: (i, 0)),
            pl.BlockSpec((k, wo), lambda i: (0, 0)),
            pl.BlockSpec((bn, 1), lambda i: (i, 0)),
        ],
        out_specs=pl.BlockSpec((bn, wo), lambda i: (i, 0)),
        out_shape=jax.ShapeDtypeStruct((n, wo), F32),
    )(x, w, scale.reshape(n, 1))


def _tc_mid(s, g, scale, b, w2, mask_n=None):
    n, w = g.shape
    wo = w2.shape[1]
    bn = _bn(n)
    nbs = (mask_n // bn) if mask_n is not None else None

    def body(s_r, g_r, sc_r, b_r, w2_r, o_r):
        sm = s_r[...]
        if mask_n is not None:
            i = pl.program_id(0)
            row = i * bn + lax.broadcasted_iota(jnp.int32, (bn, 1), 0)
            sm = jnp.where(row < mask_n, sm, 0.0)
        h = jnp.maximum(sc_r[...] * (sm + g_r[...]) + b_r[...], 0.0)
        o_r[...] = jnp.dot(h, w2_r[...],
                           preferred_element_type=F32) * sc_r[...]

    s_map = ((lambda i: (jnp.minimum(i, nbs - 1), 0)) if mask_n is not None
             else (lambda i: (i, 0)))
    return pl.pallas_call(
        body,
        grid=(n // bn,),
        in_specs=[
            pl.BlockSpec((bn, w), s_map),
            pl.BlockSpec((bn, w), lambda i: (i, 0)),
            pl.BlockSpec((bn, 1), lambda i: (i, 0)),
            pl.BlockSpec((1, w), lambda i: (0, 0)),
            pl.BlockSpec((w, wo), lambda i: (0, 0)),
        ],
        out_specs=pl.BlockSpec((bn, wo), lambda i: (i, 0)),
        out_shape=jax.ShapeDtypeStruct((n, wo), F32),
    )(s, g, scale.reshape(n, 1), b.reshape(1, w), w2)


def _tc_post(s, g, scale, b, mask_n=None):
    n, w = g.shape
    bn = _bn(n)
    nbs = (mask_n // bn) if mask_n is not None else None

    def body(s_r, g_r, sc_r, b_r, o_r):
        sm = s_r[...]
        if mask_n is not None:
            i = pl.program_id(0)
            row = i * bn + lax.broadcasted_iota(jnp.int32, (bn, 1), 0)
            sm = jnp.where(row < mask_n, sm, 0.0)
        o_r[...] = sc_r[...] * (sm + g_r[...]) + b_r[...]

    s_map = ((lambda i: (jnp.minimum(i, nbs - 1), 0)) if mask_n is not None
             else (lambda i: (i, 0)))
    return pl.pallas_call(
        body,
        grid=(n // bn,),
        in_specs=[
            pl.BlockSpec((bn, w), s_map),
            pl.BlockSpec((bn, w), lambda i: (i, 0)),
            pl.BlockSpec((bn, 1), lambda i: (i, 0)),
            pl.BlockSpec((1, w), lambda i: (0, 0)),
        ],
        out_specs=pl.BlockSpec((bn, w), lambda i: (i, 0)),
        out_shape=jax.ShapeDtypeStruct((n, w), F32),
    )(s, g, scale.reshape(n, 1), b.reshape(1, w))


def _tc_mlp(x, p, res=None, n=None):
    k = x.shape[1]
    n = x.shape[0] if n is None else n
    dh = p["W1"].shape[1]
    wo = p["W2"].shape[1]
    bn = _bn(n)

    def body(x_r, w1_r, b1_r, w2_r, b2_r, g_r, be_r, *rest):
        if res is not None:
            r_r, o_r = rest
        else:
            (o_r,) = rest
        h = jnp.maximum(jnp.dot(x_r[...], w1_r[...],
                                preferred_element_type=F32) + b1_r[...], 0.0)
        y = jnp.dot(h, w2_r[...], preferred_element_type=F32) + b2_r[...]
        mu = jnp.mean(y, axis=-1, keepdims=True)
        yc = y - mu
        var = jnp.mean(yc * yc, axis=-1, keepdims=True)
        o = yc * lax.rsqrt(var + 1e-5) * g_r[...] + be_r[...]
        if res is not None:
            o = o + r_r[...]
        o_r[...] = o

    in_specs = [
        pl.BlockSpec((bn, k), lambda i: (i, 0)),
        pl.BlockSpec((k, dh), lambda i: (0, 0)),
        pl.BlockSpec((1, dh), lambda i: (0, 0)),
        pl.BlockSpec((dh, wo), lambda i: (0, 0)),
        pl.BlockSpec((1, wo), lambda i: (0, 0)),
        pl.BlockSpec((1, wo), lambda i: (0, 0)),
        pl.BlockSpec((1, wo), lambda i: (0, 0)),
    ]
    args = [x, p["W1"], p["b1"].reshape(1, dh), p["W2"],
            p["b2"].reshape(1, wo), p["g"].reshape(1, wo),
            p["beta"].reshape(1, wo)]
    if res is not None:
        in_specs.append(pl.BlockSpec((bn, wo), lambda i: (i, 0)))
        args.append(res)
    return pl.pallas_call(
        body,
        grid=(n // bn,),
        in_specs=in_specs,
        out_specs=pl.BlockSpec((bn, wo), lambda i: (i, 0)),
        out_shape=jax.ShapeDtypeStruct((n, wo), F32),
    )(*args)


def _tc_dec(s6, g2, scale, b2p, init, init3, fcwp, fcw8, fcb):
    n = g2.shape[0]
    bn = _bn(n)

    def body(s_r, g_r, sc_r, b_r, i_r, i3_r, wp_r, w8_r, fb_r, o_r):
        d = sc_r[...] * (s_r[...] + g_r[...]) + b_r[...]
        o_r[...] = (i3_r[...] + fb_r[...]
                    + jnp.dot(d, wp_r[...], preferred_element_type=F32)
                    + jnp.dot(i_r[...], w8_r[...],
                              preferred_element_type=F32))

    return pl.pallas_call(
        body,
        grid=(n // bn,),
        in_specs=[
            pl.BlockSpec((bn, 256), lambda i: (i, 0)),
            pl.BlockSpec((bn, 256), lambda i: (i, 0)),
            pl.BlockSpec((bn, 1), lambda i: (i, 0)),
            pl.BlockSpec((1, 256), lambda i: (0, 0)),
            pl.BlockSpec((bn, 8), lambda i: (i, 0)),
            pl.BlockSpec((bn, 3), lambda i: (i, 0)),
            pl.BlockSpec((256, 3), lambda i: (0, 0)),
            pl.BlockSpec((8, 3), lambda i: (0, 0)),
            pl.BlockSpec((1, 3), lambda i: (0, 0)),
        ],
        out_specs=pl.BlockSpec((bn, 3), lambda i: (i, 0)),
        out_shape=jax.ShapeDtypeStruct((n, 3), F32),
    )(s6, g2, scale.reshape(n, 1), b2p.reshape(1, 256), init, init3,
      fcwp, fcw8, fcb.reshape(1, 3))


def kernel(params, x, edge_index, edge_attr, initial_state):
    src = edge_index[0]
    dst = edge_index[1]

    cnt = _sc_segsum(jnp.ones((NN, 256), F32), src, dst)
    deg = cnt[:NN, 0] + 1.0
    dinv_n = lax.rsqrt(deg)
    dinv_e = jnp.concatenate([dinv_n, jnp.ones((NE - NN,), F32)])

    en = params["enc_node"]
    g1 = _tc_pre(x, en["W1"], dinv_n)
    s1 = _sc_segsum(g1, src, dst)
    g2 = _tc_mid(s1, g1, dinv_n, en["b1"], en["W2"])
    s2 = _sc_segsum(g2, src, dst)
    h = _tc_post(s2, g2, dinv_n, en["b2"])

    ee = params["enc_edge"]
    a1 = _tc_pre(edge_attr, ee["W1"], dinv_e)
    s3 = _sc_segsum(a1, src, dst)
    a2 = _tc_mid(s3, a1, dinv_e, ee["b1"], ee["W2"], mask_n=NN)
    s4 = _sc_segsum(a2, src, dst)
    ea = _tc_post(s4, a2, dinv_e, ee["b2"], mask_n=NN)

    for lp in params["proc"]:
        e_rows = _tc_mlp(ea, lp["edge_mlp"])
        aggr = _sc_segsum(h, src, dst, rows=e_rows)
        h = _tc_mlp(aggr, lp["node_mlp"], res=h, n=NN)

    dc = params["dec"]
    gd1 = _tc_pre(h, dc["W1"], dinv_n)
    s5 = _sc_segsum(gd1, src, dst)
    w2p = jnp.pad(dc["W2"], ((0, 0), (0, 253)))
    gd2 = _tc_mid(s5, gd1, dinv_n, dc["b1"], w2p)
    s6 = _sc_segsum(gd2, src, dst)
    b2p = jnp.pad(dc["b2"], (0, 253))
    fcwp = jnp.pad(dc["fcW"][:3], ((0, 253), (0, 0)))
    return _tc_dec(s6, gd2, dinv_n, b2p, initial_state,
                   initial_state[:, :3], fcwp, dc["fcW"][3:], dc["fcb"])
